# baseline (speedup 1.0000x reference)
import functools

import jax
import jax.numpy as jnp
from jax import lax
from jax.experimental import pallas as pl
from jax.experimental.pallas import tpu as pltpu
from jax.experimental.pallas import tpu_sc as plsc

VOCAB = 100000
EMB = 128
BATCH = 4096
HIST = 200
B = BATCH * HIST

NC = 2
NS = 16
NW = NC * NS
B_PER_W = B // NW
CHUNK = 64
NCHUNK = B_PER_W // CHUNK
M = 5
S = M
D = 2

_vmesh = plsc.VectorSubcoreMesh(core_axis_name="c", subcore_axis_name="s")
_smesh = plsc.ScalarSubcoreMesh(axis_name="c")

_VMEM_V = pltpu.MemorySpace.VMEM @ _vmesh
_SEM_V = pltpu.MemorySpace.SEMAPHORE @ _vmesh
_SEM_S = pltpu.MemorySpace.SEMAPHORE @ _smesh


def _tec(idx_hbm, table_hbm, out_hbm, idx_v, rows_v, shared, gsem, psem,
         dsem, rsem, fsem):
    del dsem
    cid = lax.axis_index("c")
    sid = lax.axis_index("s")
    base = (sid * NC + cid) * B_PER_W

    pltpu.sync_copy(idx_hbm.at[pl.ds(base, B_PER_W)], idx_v)
    for b in range(M):
        pltpu.async_copy(
            table_hbm.at[idx_v.at[pl.ds(b * CHUNK, CHUNK)]],
            rows_v.at[b], gsem.at[b])

    def outer(i, carry):
        for u in range(M):
            c = i * M + u
            pltpu.make_async_copy(
                table_hbm.at[idx_v.at[pl.ds(u * CHUNK, CHUNK)]],
                rows_v.at[u], gsem.at[u]).wait()

            @pl.when(c >= S)
            def _slot_free():
                pl.semaphore_wait(fsem.at[u])

            pltpu.async_copy(rows_v.at[u], shared.at[sid, u], psem.at[u])

            u2 = (u - D) % M
            c2 = c - D

            @pl.when(c2 >= 0)
            def _hand_off():
                pltpu.make_async_copy(
                    rows_v.at[u2], shared.at[sid, u2], psem.at[u2]).wait()
                pl.semaphore_signal(rsem.at[sid, u2], 1, device_id={"c": cid})

                @pl.when(c2 + M < NCHUNK)
                def _refill():
                    pltpu.async_copy(
                        table_hbm.at[idx_v.at[pl.ds((c2 + M) * CHUNK, CHUNK)]],
                        rows_v.at[u2], gsem.at[u2])
        return carry

    lax.fori_loop(0, NCHUNK // M, outer, 0)

    for c2 in range(NCHUNK - D, NCHUNK):
        u2 = c2 % M
        pltpu.make_async_copy(
            rows_v.at[u2], shared.at[sid, u2], psem.at[u2]).wait()
        pl.semaphore_signal(rsem.at[sid, u2], 1, device_id={"c": cid})

    for u in range(S):
        pl.semaphore_wait(fsem.at[u])


def _scs(idx_hbm, table_hbm, out_hbm, idx_v, rows_v, shared, gsem, psem,
         dsem, rsem, fsem):
    del idx_hbm, table_hbm, idx_v, rows_v, gsem, psem
    cid = lax.axis_index("c")

    def outer(i, carry):
        for u in range(S):
            c = i * S + u
            u1 = (u - 1) % S
            for sid in range(NS):
                off = (sid * NC + cid) * B_PER_W + c * CHUNK

                @pl.when(c >= 1)
                def _retire():
                    pltpu.make_async_copy(
                        shared.at[sid, u1],
                        out_hbm.at[pl.ds(off - CHUNK, CHUNK)],
                        dsem.at[sid]).wait()
                    pl.semaphore_signal(
                        fsem.at[u1], 1, device_id={"c": cid, "s": sid})

                pl.semaphore_wait(rsem.at[sid, u])
                pltpu.async_copy(
                    shared.at[sid, u],
                    out_hbm.at[pl.ds(off, CHUNK)], dsem.at[sid])
        return carry

    lax.fori_loop(0, NCHUNK // S, outer, 0)

    ul = (NCHUNK - 1) % S
    for sid in range(NS):
        off = (sid * NC + cid) * B_PER_W + (NCHUNK - 1) * CHUNK
        pltpu.make_async_copy(
            shared.at[sid, ul],
            out_hbm.at[pl.ds(off, CHUNK)], dsem.at[sid]).wait()
        pl.semaphore_signal(fsem.at[ul], 1, device_id={"c": cid, "s": sid})


_call = pltpu


def _build():
    from jax._src.pallas import mpmd

    return mpmd.mpmd_map(
        [(_smesh, _scs), (_vmesh, _tec)],
        out_types=[jax.ShapeDtypeStruct((B, EMB), jnp.float32)],
        scratch_types=[
            _VMEM_V((B_PER_W,), jnp.int32),
            _VMEM_V((M, CHUNK, EMB), jnp.float32),
            pltpu.MemorySpace.VMEM_SHARED((NS, S, CHUNK, EMB), jnp.float32),
            (pltpu.MemorySpace.SEMAPHORE @ _vmesh)(
                (M,), pltpu.SemaphoreType.DMA.dtype),
            (pltpu.MemorySpace.SEMAPHORE @ _vmesh)(
                (M,), pltpu.SemaphoreType.DMA.dtype),
            (pltpu.MemorySpace.SEMAPHORE @ _smesh)(
                (NS,), pltpu.SemaphoreType.DMA.dtype),
            (pltpu.MemorySpace.SEMAPHORE @ _smesh)(
                (NS, S), pltpu.SemaphoreType.REGULAR.dtype),
            (pltpu.MemorySpace.SEMAPHORE @ _vmesh)(
                (S,), pltpu.SemaphoreType.REGULAR.dtype),
        ],
    )


_gather = _build()


def kernel(inputs, embedding):
    idx = inputs.reshape(-1).astype(jnp.int32)
    (out,) = _gather(idx, embedding)
    return out.reshape(BATCH, HIST, EMB)


# speedup vs baseline: 1.0146x; 1.0146x over previous
import functools

import jax
import jax.numpy as jnp
from jax import lax
from jax.experimental import pallas as pl
from jax.experimental.pallas import tpu as pltpu
from jax.experimental.pallas import tpu_sc as plsc

VOCAB = 100000
EMB = 128
BATCH = 4096
HIST = 200
B = BATCH * HIST

NC = 2
NS = 16
NW = NC * NS
B_PER_W = B // NW
CHUNK = 80
NCHUNK = B_PER_W // CHUNK
M = 4
S = 4
D = 2

_mesh = plsc.VectorSubcoreMesh(core_axis_name="c", subcore_axis_name="s")


@functools.partial(
    pl.kernel,
    mesh=_mesh,
    out_type=jax.ShapeDtypeStruct((B, EMB), jnp.float32),
    scratch_types=[
        pltpu.VMEM((B_PER_W,), jnp.int32),
        pltpu.VMEM((M, CHUNK, EMB), jnp.float32),
        pltpu.VMEM_SHARED((NS, S, CHUNK, EMB), jnp.float32),
        pltpu.SemaphoreType.DMA((M,)),
        pltpu.SemaphoreType.DMA((S,)),
        pltpu.SemaphoreType.DMA((S,)),
    ],
)
def _gather(idx_hbm, table_hbm, out_hbm, idx_v, rows_v, shared, gsem, psem,
            osem):
    cid = lax.axis_index("c")
    sid = lax.axis_index("s")
    wid = sid * NC + cid
    base = wid * B_PER_W

    pltpu.sync_copy(idx_hbm.at[pl.ds(base, B_PER_W)], idx_v)
    for b in range(M):
        pltpu.async_copy(
            table_hbm.at[idx_v.at[pl.ds(b * CHUNK, CHUNK)]],
            rows_v.at[b], gsem.at[b])

    def outer(i, carry):
        for u in range(M):
            c = i * M + u
            off = base + c * CHUNK
            pltpu.make_async_copy(
                table_hbm.at[idx_v.at[pl.ds(u * CHUNK, CHUNK)]],
                rows_v.at[u], gsem.at[u]).wait()

            @pl.when(c >= S)
            def _slot_free():
                pltpu.make_async_copy(
                    shared.at[sid, u],
                    out_hbm.at[pl.ds(off - S * CHUNK, CHUNK)],
                    osem.at[u]).wait()

            pltpu.async_copy(rows_v.at[u], shared.at[sid, u], psem.at[u])

            u2 = (u - D) % M
            c2 = c - D

            @pl.when(c2 >= 0)
            def _drain():
                off2 = base + c2 * CHUNK
                pltpu.make_async_copy(
                    rows_v.at[u2], shared.at[sid, u2], psem.at[u2]).wait()
                pltpu.async_copy(
                    shared.at[sid, u2],
                    out_hbm.at[pl.ds(off2, CHUNK)], osem.at[u2])

                @pl.when(c2 + M < NCHUNK)
                def _refill():
                    pltpu.async_copy(
                        table_hbm.at[idx_v.at[pl.ds((c2 + M) * CHUNK, CHUNK)]],
                        rows_v.at[u2], gsem.at[u2])
        return carry

    lax.fori_loop(0, NCHUNK // M, outer, 0)

    for c2 in range(NCHUNK - D, NCHUNK):
        u2 = c2 % M
        off2 = base + c2 * CHUNK
        pltpu.make_async_copy(
            rows_v.at[u2], shared.at[sid, u2], psem.at[u2]).wait()
        pltpu.async_copy(
            shared.at[sid, u2], out_hbm.at[pl.ds(off2, CHUNK)], osem.at[u2])

    for c2 in range(NCHUNK - S, NCHUNK):
        u2 = c2 % S
        off2 = base + c2 * CHUNK
        pltpu.make_async_copy(
            shared.at[sid, u2], out_hbm.at[pl.ds(off2, CHUNK)],
            osem.at[u2]).wait()


def kernel(inputs, embedding):
    idx = inputs.reshape(-1).astype(jnp.int32)
    out = _gather(idx, embedding)
    return out.reshape(BATCH, HIST, EMB)
